# baseline (speedup 1.0000x reference)
import functools

import jax
import jax.numpy as jnp
from jax.experimental import pallas as pl
from jax.experimental.pallas import tpu as pltpu

N_NODES = 10000
N_EDGES = 160000
K_SIMPLE = 15
K_TRI = 20
BLK = 1000
TNT_STATIC = 1000


def _mlp_body(r_ref, wm1_ref, bm1_ref, wm2_ref, bm2_ref, pinit_ref, out_ref):
    r = r_ref[...]
    h = jnp.maximum(r @ wm1_ref[...] + bm1_ref[...], 0.0)
    tb = r.shape[0] // K_TRI
    pooled = h.reshape(tb, K_TRI, 128).mean(axis=1)
    logits = (pooled @ wm2_ref[...]) + bm2_ref[0, 0]
    out_ref[...] = jax.nn.sigmoid(logits) * pinit_ref[...]


def _mlp_stage(r, Wm1, bm1, Wm2, bm2, p_init):
    T = r.shape[0]
    TB = 2000
    r8 = jnp.concatenate([r, jnp.zeros((T, K_TRI, 2), jnp.float32)], axis=-1)
    r8 = r8.reshape(T * K_TRI, 8)
    Wm1p = jnp.concatenate([Wm1, jnp.zeros((2, 128), jnp.float32)], axis=0)
    out = pl.pallas_call(
        _mlp_body,
        grid=(T // TB,),
        in_specs=[
            pl.BlockSpec((TB * K_TRI, 8), lambda i: (i, 0)),
            pl.BlockSpec((8, 128), lambda i: (0, 0)),
            pl.BlockSpec((1, 128), lambda i: (0, 0)),
            pl.BlockSpec((128, 1), lambda i: (0, 0)),
            pl.BlockSpec((1, 1), lambda i: (0, 0)),
            pl.BlockSpec((TB, 1), lambda i: (i, 0)),
        ],
        out_specs=pl.BlockSpec((TB, 1), lambda i: (i, 0)),
        out_shape=jax.ShapeDtypeStruct((T, 1), jnp.float32),
    )(r8, Wm1p, bm1.reshape(1, 128), Wm2, bm2.reshape(1, 1), p_init.reshape(T, 1))
    return out.reshape(T)


def kernel(x, edge_index, Wg1, Wg2, Wd1, Wd2, Wm1, bm1, Wm2, bm2, target_number_triangles):
    N = x.shape[0]
    src, dst = edge_index[0], edge_index[1]
    msg = jax.nn.relu((x[src] - x[dst]) @ Wg1)
    agg = jax.ops.segment_max(msg, dst, num_segments=N)
    agg = jnp.maximum(agg, 0.0)
    scores = (agg @ Wg2).squeeze(-1)
    Pn = min(N, 3 * TNT_STATIC)
    sel = jax.lax.top_k(jax.lax.stop_gradient(scores), Pn)[1]
    nodes = x[sel] * jax.nn.sigmoid(scores[sel])[:, None]
    d2 = jnp.sum(nodes ** 2, axis=1)
    dist = d2[:, None] + d2[None, :] - 2.0 * nodes @ nodes.T
    knn = jax.lax.top_k(jax.lax.stop_gradient(-dist), K_SIMPLE + 1)[1][:, 1:]
    rows = jnp.repeat(jnp.arange(Pn), K_SIMPLE)
    cols = knn.reshape(-1)
    A = jnp.zeros((Pn, Pn), dtype=jnp.float32).at[rows, cols].set(1.0)
    diff = nodes[cols] - nodes[rows]
    h = jax.nn.relu(diff @ Wd1)
    score_edge = (h @ Wd2).squeeze(-1)
    m = jax.ops.segment_max(jax.lax.stop_gradient(score_edge), rows, num_segments=Pn)
    ex = jnp.exp(score_edge - m[rows])
    den = jax.ops.segment_sum(ex, rows, num_segments=Pn)
    S_e = ex / (den[rows] + 1e-12)
    S = jnp.zeros((Pn, Pn), dtype=jnp.float32).at[rows, cols].set(S_e)
    A_sym = jnp.maximum(A, A.T)
    P_adj = 0.5 * (S + S.T) * A_sym
    a = jnp.repeat(jnp.arange(Pn), K_SIMPLE - 1)
    b = knn[:, :K_SIMPLE - 1].reshape(-1)
    c = knn[:, 1:K_SIMPLE].reshape(-1)
    tri_ids = jnp.stack([a, b, c], axis=1)
    T = tri_ids.shape[0]
    triangles = nodes[tri_ids]
    p_init = (P_adj[a, b] * P_adj[b, c] * P_adj[c, a] + 1e-12) ** (1.0 / 3.0)
    bary = triangles.mean(axis=1)
    bnorm = jnp.sum(bary ** 2, axis=1)

    def blk(qi):
        q = jax.lax.dynamic_slice(bary, (qi * BLK, 0), (BLK, 3))
        qn = jax.lax.dynamic_slice(bnorm, (qi * BLK,), (BLK,))
        d = qn[:, None] + bnorm[None, :] - 2.0 * q @ bary.T
        return jax.lax.top_k(jax.lax.stop_gradient(-d), K_TRI)[1]

    nb = T // BLK
    idx_n = jax.lax.map(blk, jnp.arange(nb)).reshape(T, K_TRI).astype(jnp.int32)
    v0, v1, v2 = triangles[:, 0], triangles[:, 1], triangles[:, 2]
    nrm = jnp.cross(v1 - v0, v2 - v0)
    nrm = nrm / (jnp.linalg.norm(nrm, axis=1, keepdims=True) + 1e-8)
    bdiff = bary[idx_n] - bary[:, None, :]
    r = jnp.concatenate([bdiff, nrm[idx_n]], axis=-1)
    final = _mlp_stage(r, Wm1, bm1, Wm2, bm2, p_init)
    top_p, top_i = jax.lax.top_k(final, TNT_STATIC)
    return triangles[top_i]
